# baseline (speedup 1.0000x reference)
import functools

import jax
import jax.numpy as jnp
from jax import lax
from jax.experimental import pallas as pl
from jax.experimental.pallas import tpu as pltpu
from jax.experimental.pallas import tpu_sc as plsc

_NC = 2
_NS = 16
_NW = _NC * _NS
_B = 16384
_D = 64
_BPW = _B // _NW
_CH = 128
_NCH = _BPW // _CH
_EPS = 1e-6


def _vsqrt(x):
    xs = jnp.maximum(x, jnp.float32(1e-35))
    i = lax.bitcast_convert_type(xs, jnp.int32)
    i = jnp.int32(0x5F3759DF) - lax.shift_right_logical(i, 1)
    y = lax.bitcast_convert_type(i, jnp.float32)
    for _ in range(3):
        y = y * (jnp.float32(1.5) - jnp.float32(0.5) * xs * y * y)
    return xs * y


_mesh = plsc.VectorSubcoreMesh(core_axis_name="c", subcore_axis_name="s")


@functools.partial(
    pl.kernel,
    out_type=jax.ShapeDtypeStruct((_NW, 16), jnp.float32),
    mesh=_mesh,
    compiler_params=pltpu.CompilerParams(
        needs_layout_passes=False, use_tc_tiling_on_sc=False),
    scratch_types=[
        pltpu.VMEM((_NCH, _CH), jnp.int32),
        pltpu.VMEM((_BPW, _D), jnp.float32),
        pltpu.VMEM((_BPW, _D), jnp.float32),
        pltpu.VMEM((16,), jnp.float32),
        pltpu.SemaphoreType.DMA,
    ],
)
def _center_loss_sc(table, idx2d, batch, out, idx_v, rows_v, batch_v,
                    acc_v, sem):
    wid = lax.axis_index("s") * _NC + lax.axis_index("c")

    pltpu.sync_copy(idx2d.at[pl.ds(wid * _NCH, _NCH)], idx_v)
    copies = [
        pltpu.async_copy(
            table.at[idx_v.at[j]],
            rows_v.at[pl.ds(j * _CH, _CH)],
            sem,
        )
        for j in range(_NCH)
    ]
    copies.append(
        pltpu.async_copy(batch.at[pl.ds(wid * _BPW, _BPW)], batch_v, sem))
    for cp in copies:
        cp.wait()

    lanes = lax.iota(jnp.int32, 16)

    def blk_body(blk, acc):
        merged = jnp.zeros((16,), jnp.float32)
        for i in range(16):
            r = blk * 16 + i
            sq = None
            for k in range(_D // 16):
                g = rows_v[r, pl.ds(k * 16, 16)]
                b = batch_v[r, pl.ds(k * 16, 16)]
                d = g - b + jnp.float32(_EPS)
                m = d * d
                sq = m if sq is None else sq + m
            tot = jnp.sum(sq)
            merged = jnp.where(lanes == i, tot, merged)
        return acc + _vsqrt(merged)

    acc = lax.fori_loop(0, _BPW // 16, blk_body, jnp.zeros((16,), jnp.float32))
    acc_v[...] = acc
    pltpu.sync_copy(acc_v, out.at[wid])


def kernel(center_list, batch_center_vecs, gt_labels):
    idx2d = gt_labels.reshape(_NW * _NCH, _CH)
    partials = _center_loss_sc(center_list, idx2d, batch_center_vecs)
    return jnp.sum(partials) / jnp.float32(_B)
